# baseline (speedup 1.0000x reference)
import functools

import jax
import jax.numpy as jnp
from jax import lax
from jax.experimental import pallas as pl
from jax.experimental.pallas import tpu as pltpu
from jax.experimental.pallas import tpu_sc as plsc

B = 64
S = 512
H = 768
EPS = 1e-12
NC = 2
NS = 16
NW = NC * NS
SPT = S // NW
HV = H // 16


def _rsqrt_vec(x):
    half = jnp.full((16,), 0.5, jnp.float32)
    three_half = jnp.full((16,), 1.5, jnp.float32)
    i = plsc.bitcast(x, jnp.int32)
    i = jnp.full((16,), 0x5F3759DF, jnp.int32) - lax.shift_right_arithmetic(i, 1)
    y = plsc.bitcast(i, jnp.float32)
    hx = half * x
    for _ in range(3):
        y = y * (three_half - hx * y * y)
    return y


def _tile_body(ids_hbm, word_hbm, pos_hbm, tok_hbm, g_hbm, bt_hbm, out_hbm,
               idx_v, bias_t, buf0, buf1, g_smem, b_smem, sem0, sem1):
    c = lax.axis_index("c")
    s_ = lax.axis_index("s")
    w = s_ * NC + c

    lanes = lax.iota(jnp.int32, 16)
    zeros_i = jnp.zeros((16,), jnp.int32)
    ones_i = jnp.full((16,), 1, jnp.int32)

    pltpu.sync_copy(ids_hbm.at[w], idx_v)
    pltpu.sync_copy(pos_hbm.at[pl.ds(w * SPT, SPT)], buf0)
    pltpu.sync_copy(tok_hbm.at[0], buf1.at[0])

    def _tr(h, colv):
        pcol = plsc.load_gather(buf0, [lanes, colv])
        tcol = plsc.load_gather(buf1, [zeros_i, colv])
        bias_t[h, :] = pcol + tcol
        return colv + ones_i
    lax.fori_loop(0, H, _tr, zeros_i, unroll=4)

    pltpu.sync_copy(g_hbm, buf0.at[0])
    pltpu.sync_copy(bt_hbm, buf0.at[1])

    def _gb(j, carry):
        base = j * 16
        gv = buf0[0, pl.ds(base, 16)]
        bv = buf0[1, pl.ds(base, 16)]
        for k in range(16):
            g_smem[base + k] = gv[k]
            b_smem[base + k] = bv[k]
        return carry
    lax.fori_loop(0, HV, _gb, 0)

    one_over_h = jnp.full((16,), 1.0 / H, jnp.float32)
    eps_v = jnp.full((16,), EPS, jnp.float32)

    def _ln16(buf):
        def _p1(h, carry):
            colv, sumv, sqv = carry
            x = plsc.load_gather(buf, [lanes, colv])
            xb = x + bias_t[h, :]
            plsc.store_scatter(buf, [lanes, colv], xb)
            return colv + ones_i, sumv + xb, sqv + xb * xb
        zf = jnp.zeros((16,), jnp.float32)
        _, sumv, sqv = lax.fori_loop(0, H, _p1, (zeros_i, zf, zf), unroll=8)
        mean = sumv * one_over_h
        var = sqv * one_over_h - mean * mean
        istd = _rsqrt_vec(var + eps_v)

        def _p2(h, colv):
            xb = plsc.load_gather(buf, [lanes, colv])
            t = (xb - mean) * istd
            o = t * g_smem[h] + b_smem[h]
            plsc.store_scatter(buf, [lanes, colv], o)
            return colv + ones_i
        lax.fori_loop(0, H, _p2, zeros_i, unroll=8)

    def _start_gather(b, buf, sem):
        pltpu.async_copy(word_hbm.at[idx_v.at[b]], buf, sem)

    def _wait_gather(b, buf, sem):
        pltpu.make_async_copy(word_hbm.at[idx_v.at[b]], buf, sem).wait()

    def _finish(b, buf):
        _ln16(buf)
        pltpu.sync_copy(buf, out_hbm.at[pl.ds(b * S + w * SPT, SPT)])

    _start_gather(0, buf0, sem0)

    def _b_step(i, carry):
        b0 = i * 2
        _start_gather(b0 + 1, buf1, sem1)
        _wait_gather(b0, buf0, sem0)
        _finish(b0, buf0)

        @pl.when(i < B // 2 - 1)
        def _():
            _start_gather(b0 + 2, buf0, sem0)
        _wait_gather(b0 + 1, buf1, sem1)
        _finish(b0 + 1, buf1)
        return carry
    lax.fori_loop(0, B // 2, _b_step, 0)


_sc_call = functools.partial(
    pl.kernel,
    out_type=jax.ShapeDtypeStruct((B * S, H), jnp.float32),
    mesh=plsc.VectorSubcoreMesh(core_axis_name="c", subcore_axis_name="s"),
    compiler_params=pltpu.CompilerParams(needs_layout_passes=False),
    scratch_types=[
        pltpu.VMEM((B, SPT), jnp.int32),
        pltpu.VMEM((H, SPT), jnp.float32),
        pltpu.VMEM((SPT, H), jnp.float32),
        pltpu.VMEM((SPT, H), jnp.float32),
        pltpu.SMEM((H,), jnp.float32),
        pltpu.SMEM((H,), jnp.float32),
        pltpu.SemaphoreType.DMA,
        pltpu.SemaphoreType.DMA,
    ],
)(_tile_body)


def kernel(input_ids, word_emb, pos_emb, tok_emb, gamma, beta):
    ids = input_ids.astype(jnp.int32).reshape(B, NW, SPT).transpose(1, 0, 2)
    out = _sc_call(ids, word_emb, pos_emb, tok_emb, gamma, beta)
    return out.reshape(B, S, H)


# speedup vs baseline: 2.2983x; 2.2983x over previous
import functools

import jax
import jax.numpy as jnp
from jax import lax
from jax.experimental import pallas as pl
from jax.experimental.pallas import tpu as pltpu
from jax.experimental.pallas import tpu_sc as plsc

B = 64
S = 512
H = 768
EPS = 1e-12
NC = 2
NS = 16
NW = NC * NS
SPT = S // NW
HV = H // 16


def _rsqrt_vec(x):
    half = jnp.full((16,), 0.5, jnp.float32)
    three_half = jnp.full((16,), 1.5, jnp.float32)
    i = plsc.bitcast(x, jnp.int32)
    i = jnp.full((16,), 0x5F3759DF, jnp.int32) - lax.shift_right_arithmetic(i, 1)
    y = plsc.bitcast(i, jnp.float32)
    hx = half * x
    for _ in range(3):
        y = y * (three_half - hx * y * y)
    return y


def _tile_body(ids_hbm, word_hbm, pos_hbm, tok_hbm, g_hbm, bt_hbm, out_hbm,
               idx_v, bias_t, g_t, b_t, gb_v, buf0, buf1, sem0, sem1):
    c = lax.axis_index("c")
    s_ = lax.axis_index("s")
    w = s_ * NC + c

    lanes = lax.iota(jnp.int32, 16)
    zeros_i = jnp.zeros((16,), jnp.int32)
    ones_i = jnp.full((16,), 1, jnp.int32)

    diags = [(lanes + k) & 15 for k in range(16)]

    pltpu.sync_copy(ids_hbm.at[w], idx_v)
    pltpu.sync_copy(pos_hbm.at[pl.ds(w * SPT, SPT)], buf0)
    pltpu.sync_copy(tok_hbm.at[0], gb_v.at[0])

    def _fold(r, carry):
        for j in range(HV):
            d = pl.ds(j * 16, 16)
            buf0[r, d] = buf0[r, d] + gb_v[0, d]
        return carry
    lax.fori_loop(0, SPT, _fold, 0)

    pltpu.sync_copy(g_hbm, gb_v.at[0])
    pltpu.sync_copy(bt_hbm, gb_v.at[1])

    def _tr(hg, carry):
        basev = lax.broadcast_in_dim(hg * 16, (16,), ())
        for k in range(16):
            col = basev + diags[k]
            h = hg * 16 + k
            bias_t[h, :] = plsc.load_gather(buf0, [lanes, col])
            g_t[h, :] = plsc.load_gather(gb_v, [zeros_i, col])
            b_t[h, :] = plsc.load_gather(gb_v, [ones_i, col])
        return carry
    lax.fori_loop(0, HV, _tr, 0)

    one_over_h = jnp.full((16,), 1.0 / H, jnp.float32)
    eps_v = jnp.full((16,), EPS, jnp.float32)

    def _ln16(buf):
        def _p1(hg, carry):
            s0, s1, q0, q1 = carry
            basev = lax.broadcast_in_dim(hg * 16, (16,), ())
            for k in range(16):
                col = basev + diags[k]
                x = plsc.load_gather(buf, [lanes, col])
                xb = x + bias_t[hg * 16 + k, :]
                plsc.store_scatter(buf, [lanes, col], xb)
                if k & 1:
                    s1 = s1 + xb
                    q1 = q1 + xb * xb
                else:
                    s0 = s0 + xb
                    q0 = q0 + xb * xb
            return s0, s1, q0, q1
        zf = jnp.zeros((16,), jnp.float32)
        s0, s1, q0, q1 = lax.fori_loop(0, HV, _p1, (zf, zf, zf, zf))
        mean = (s0 + s1) * one_over_h
        var = (q0 + q1) * one_over_h - mean * mean
        istd = _rsqrt_vec(var + eps_v)

        def _p2(hg, carry):
            basev = lax.broadcast_in_dim(hg * 16, (16,), ())
            for k in range(16):
                col = basev + diags[k]
                h = hg * 16 + k
                xb = plsc.load_gather(buf, [lanes, col])
                t = (xb - mean) * istd
                o = t * g_t[h, :] + b_t[h, :]
                plsc.store_scatter(buf, [lanes, col], o)
            return carry
        lax.fori_loop(0, HV, _p2, 0)

    def _start_gather(b, buf, sem):
        pltpu.async_copy(word_hbm.at[idx_v.at[b]], buf, sem)

    def _wait_gather(b, buf, sem):
        pltpu.make_async_copy(word_hbm.at[idx_v.at[b]], buf, sem).wait()

    def _finish(b, buf):
        _ln16(buf)
        pltpu.sync_copy(buf, out_hbm.at[pl.ds(b * S + w * SPT, SPT)])

    _start_gather(0, buf0, sem0)

    def _b_step(i, carry):
        b0 = i * 2
        _start_gather(b0 + 1, buf1, sem1)
        _wait_gather(b0, buf0, sem0)
        _finish(b0, buf0)

        @pl.when(i < B // 2 - 1)
        def _():
            _start_gather(b0 + 2, buf0, sem0)
        _wait_gather(b0 + 1, buf1, sem1)
        _finish(b0 + 1, buf1)
        return carry
    lax.fori_loop(0, B // 2, _b_step, 0)


_sc_call = functools.partial(
    pl.kernel,
    out_type=jax.ShapeDtypeStruct((B * S, H), jnp.float32),
    mesh=plsc.VectorSubcoreMesh(core_axis_name="c", subcore_axis_name="s"),
    compiler_params=pltpu.CompilerParams(
        needs_layout_passes=False, use_tc_tiling_on_sc=False),
    scratch_types=[
        pltpu.VMEM((B, SPT), jnp.int32),
        pltpu.VMEM((H, SPT), jnp.float32),
        pltpu.VMEM((H, SPT), jnp.float32),
        pltpu.VMEM((H, SPT), jnp.float32),
        pltpu.VMEM((2, H), jnp.float32),
        pltpu.VMEM((SPT, H), jnp.float32),
        pltpu.VMEM((SPT, H), jnp.float32),
        pltpu.SemaphoreType.DMA,
        pltpu.SemaphoreType.DMA,
    ],
)(_tile_body)


def kernel(input_ids, word_emb, pos_emb, tok_emb, gamma, beta):
    ids = input_ids.astype(jnp.int32).reshape(B, NW, SPT).transpose(1, 0, 2)
    out = _sc_call(ids, word_emb, pos_emb, tok_emb, gamma, beta)
    return out.reshape(B, S, H)


# speedup vs baseline: 3.4616x; 1.5062x over previous
import functools

import jax
import jax.numpy as jnp
from jax import lax
from jax.experimental import pallas as pl
from jax.experimental.pallas import tpu as pltpu
from jax.experimental.pallas import tpu_sc as plsc

B = 64
S = 512
H = 768
EPS = 1e-12
NC = 2
NS = 16
NW = NC * NS
SPT = S // NW
HV = H // 16

_GATHER_DNUMS = lax.GatherDimensionNumbers(
    offset_dims=(), collapsed_slice_dims=(0,), start_index_map=(0,))


def _lane_sum(x):
    lanes = lax.iota(jnp.int32, 16)
    for sh in (8, 4, 2, 1):
        idx = (lanes + sh) & 15
        x = x + lax.gather(x, idx[:, None], _GATHER_DNUMS, (1,),
                           mode=lax.GatherScatterMode.PROMISE_IN_BOUNDS)
    return x


def _rsqrt_vec(x):
    half = jnp.full((16,), 0.5, jnp.float32)
    three_half = jnp.full((16,), 1.5, jnp.float32)
    i = plsc.bitcast(x, jnp.int32)
    i = jnp.full((16,), 0x5F3759DF, jnp.int32) - lax.shift_right_arithmetic(i, 1)
    y = plsc.bitcast(i, jnp.float32)
    hx = half * x
    for _ in range(2):
        y = y * (three_half - hx * y * y)
    return y


def _tile_body(ids_hbm, word_hbm, pos_hbm, tok_hbm, g_hbm, bt_hbm, out_hbm,
               idx_v, bias_v, tok_v, gamma_v, beta_v, buf0, buf1, sem0, sem1):
    c = lax.axis_index("c")
    s_ = lax.axis_index("s")
    w = s_ * NC + c

    pltpu.sync_copy(ids_hbm.at[w], idx_v)
    pltpu.sync_copy(pos_hbm.at[pl.ds(w * SPT, SPT)], bias_v)
    pltpu.sync_copy(tok_hbm.at[0], tok_v)
    pltpu.sync_copy(g_hbm, gamma_v)
    pltpu.sync_copy(bt_hbm, beta_v)

    def _fold(sl, carry):
        for j in range(HV):
            d = pl.ds(j * 16, 16)
            bias_v[sl, d] = bias_v[sl, d] + tok_v[d]
        return carry
    lax.fori_loop(0, SPT, _fold, 0)

    one_over_h = jnp.full((16,), 1.0 / H, jnp.float32)
    eps_v = jnp.full((16,), EPS, jnp.float32)

    def _one_row(buf, r):
        s0 = s1 = q0 = q1 = jnp.zeros((16,), jnp.float32)
        for j in range(HV):
            d = pl.ds(j * 16, 16)
            xb = buf[r, d] + bias_v[r, d]
            buf[r, d] = xb
            if j & 1:
                s1 = s1 + xb
                q1 = q1 + xb * xb
            else:
                s0 = s0 + xb
                q0 = q0 + xb * xb
        mean = _lane_sum(s0 + s1) * one_over_h
        var = _lane_sum(q0 + q1) * one_over_h - mean * mean
        istd = _rsqrt_vec(var + eps_v)
        for j in range(HV):
            d = pl.ds(j * 16, 16)
            t = (buf[r, d] - mean) * istd
            buf[r, d] = t * gamma_v[d] + beta_v[d]

    def _ln16(buf):
        def _r_step(r2, carry):
            _one_row(buf, r2 * 2)
            _one_row(buf, r2 * 2 + 1)
            return carry
        lax.fori_loop(0, SPT // 2, _r_step, 0)

    def _start_gather(b, buf, sem):
        pltpu.async_copy(word_hbm.at[idx_v.at[b]], buf, sem)

    def _wait_gather(b, buf, sem):
        pltpu.make_async_copy(word_hbm.at[idx_v.at[b]], buf, sem).wait()

    def _finish(b, buf):
        _ln16(buf)
        pltpu.sync_copy(buf, out_hbm.at[pl.ds(b * S + w * SPT, SPT)])

    _start_gather(0, buf0, sem0)

    def _b_step(i, carry):
        b0 = i * 2
        _start_gather(b0 + 1, buf1, sem1)
        _wait_gather(b0, buf0, sem0)
        _finish(b0, buf0)

        @pl.when(i < B // 2 - 1)
        def _():
            _start_gather(b0 + 2, buf0, sem0)
        _wait_gather(b0 + 1, buf1, sem1)
        _finish(b0 + 1, buf1)
        return carry
    lax.fori_loop(0, B // 2, _b_step, 0)


_sc_call = functools.partial(
    pl.kernel,
    out_type=jax.ShapeDtypeStruct((B * S, H), jnp.float32),
    mesh=plsc.VectorSubcoreMesh(core_axis_name="c", subcore_axis_name="s"),
    compiler_params=pltpu.CompilerParams(
        needs_layout_passes=False, use_tc_tiling_on_sc=False),
    scratch_types=[
        pltpu.VMEM((B, SPT), jnp.int32),
        pltpu.VMEM((SPT, H), jnp.float32),
        pltpu.VMEM((H,), jnp.float32),
        pltpu.VMEM((H,), jnp.float32),
        pltpu.VMEM((H,), jnp.float32),
        pltpu.VMEM((SPT, H), jnp.float32),
        pltpu.VMEM((SPT, H), jnp.float32),
        pltpu.SemaphoreType.DMA,
        pltpu.SemaphoreType.DMA,
    ],
)(_tile_body)


def kernel(input_ids, word_emb, pos_emb, tok_emb, gamma, beta):
    ids = input_ids.astype(jnp.int32).reshape(B, NW, SPT).transpose(1, 0, 2)
    out = _sc_call(ids, word_emb, pos_emb, tok_emb, gamma, beta)
    return out.reshape(B, S, H)


# speedup vs baseline: 4.9727x; 1.4366x over previous
import functools

import jax
import jax.numpy as jnp
from jax import lax
from jax.experimental import pallas as pl
from jax.experimental.pallas import tpu as pltpu
from jax.experimental.pallas import tpu_sc as plsc

B = 64
S = 512
H = 768
EPS = 1e-12
NC = 2
NS = 16
NW = NC * NS
SPT = S // NW
HV = H // 16

_GATHER_DNUMS = lax.GatherDimensionNumbers(
    offset_dims=(), collapsed_slice_dims=(0,), start_index_map=(0,))


def _lane_sum(x):
    lanes = lax.iota(jnp.int32, 16)
    for sh in (8, 4, 2, 1):
        idx = (lanes + sh) & 15
        x = x + lax.gather(x, idx[:, None], _GATHER_DNUMS, (1,),
                           mode=lax.GatherScatterMode.PROMISE_IN_BOUNDS)
    return x


def _rsqrt_vec(x):
    half = jnp.full((16,), 0.5, jnp.float32)
    three_half = jnp.full((16,), 1.5, jnp.float32)
    i = plsc.bitcast(x, jnp.int32)
    i = jnp.full((16,), 0x5F3759DF, jnp.int32) - lax.shift_right_arithmetic(i, 1)
    y = plsc.bitcast(i, jnp.float32)
    hx = half * x
    for _ in range(2):
        y = y * (three_half - hx * y * y)
    return y


def _tile_body(ids_hbm, word_hbm, pos_hbm, tok_hbm, g_hbm, bt_hbm, out_hbm,
               idx_v, bias_v, tok_v, gamma_v, beta_v, buf0, buf1, sem0, sem1):
    c = lax.axis_index("c")
    s_ = lax.axis_index("s")
    w = s_ * NC + c

    pltpu.sync_copy(ids_hbm.at[w], idx_v)
    pltpu.sync_copy(pos_hbm.at[pl.ds(w * SPT, SPT)], bias_v)
    pltpu.sync_copy(tok_hbm.at[0], tok_v)
    pltpu.sync_copy(g_hbm, gamma_v)
    pltpu.sync_copy(bt_hbm, beta_v)

    def _fold(sl, carry):
        for j in range(HV):
            d = pl.ds(j * 16, 16)
            bias_v[sl, d] = bias_v[sl, d] + tok_v[d]
        return carry
    lax.fori_loop(0, SPT, _fold, 0)

    one_over_h = jnp.full((16,), 1.0 / H, jnp.float32)
    eps_v = jnp.full((16,), EPS, jnp.float32)

    def _one_row(buf, r):
        s0 = s1 = q0 = q1 = jnp.zeros((16,), jnp.float32)
        for j in range(HV):
            d = pl.ds(j * 16, 16)
            xb = buf[r, d] + bias_v[r, d]
            buf[r, d] = xb
            if j & 1:
                s1 = s1 + xb
                q1 = q1 + xb * xb
            else:
                s0 = s0 + xb
                q0 = q0 + xb * xb
        mean = _lane_sum(s0 + s1) * one_over_h
        var = _lane_sum(q0 + q1) * one_over_h - mean * mean
        istd = _rsqrt_vec(var + eps_v)
        for j in range(HV):
            d = pl.ds(j * 16, 16)
            t = (buf[r, d] - mean) * istd
            buf[r, d] = t * gamma_v[d] + beta_v[d]

    def _ln16(buf):
        def _r_step(r, carry):
            _one_row(buf, r)
            return carry
        lax.fori_loop(0, SPT, _r_step, 0)

    def _start_gather(b, buf, sem):
        pltpu.async_copy(word_hbm.at[idx_v.at[b]], buf, sem)

    def _wait_gather(b, buf, sem):
        pltpu.make_async_copy(word_hbm.at[idx_v.at[b]], buf, sem).wait()

    def _finish(b, buf):
        _ln16(buf)
        pltpu.sync_copy(buf, out_hbm.at[pl.ds(b * S + w * SPT, SPT)])

    _start_gather(0, buf0, sem0)

    def _b_step(i, carry):
        b0 = i * 2
        _start_gather(b0 + 1, buf1, sem1)
        _wait_gather(b0, buf0, sem0)
        _finish(b0, buf0)

        @pl.when(i < B // 2 - 1)
        def _():
            _start_gather(b0 + 2, buf0, sem0)
        _wait_gather(b0 + 1, buf1, sem1)
        _finish(b0 + 1, buf1)
        return carry
    lax.fori_loop(0, B // 2, _b_step, 0)


_sc_call = functools.partial(
    pl.kernel,
    out_type=jax.ShapeDtypeStruct((B * S, H), jnp.float32),
    mesh=plsc.VectorSubcoreMesh(core_axis_name="c", subcore_axis_name="s"),
    compiler_params=pltpu.CompilerParams(needs_layout_passes=False),
    scratch_types=[
        pltpu.VMEM((B, SPT), jnp.int32),
        pltpu.VMEM((SPT, H), jnp.float32),
        pltpu.VMEM((H,), jnp.float32),
        pltpu.VMEM((H,), jnp.float32),
        pltpu.VMEM((H,), jnp.float32),
        pltpu.VMEM((SPT, H), jnp.float32),
        pltpu.VMEM((SPT, H), jnp.float32),
        pltpu.SemaphoreType.DMA,
        pltpu.SemaphoreType.DMA,
    ],
)(_tile_body)


def kernel(input_ids, word_emb, pos_emb, tok_emb, gamma, beta):
    ids = input_ids.astype(jnp.int32).reshape(B, NW, SPT).transpose(1, 0, 2)
    out = _sc_call(ids, word_emb, pos_emb, tok_emb, gamma, beta)
    return out.reshape(B, S, H)
